# baseline (speedup 1.0000x reference)
import functools

import jax
import jax.numpy as jnp
from jax import lax
from jax.experimental import pallas as pl
from jax.experimental.pallas import tpu as pltpu
from jax.experimental.pallas import tpu_sc as plsc

_LANES = 16


def _make_sc_add(B, S, D, NC, NS, CH, UNROLL):
    NW = NC * NS
    rows_per_w = S // NW
    n_chunks = rows_per_w // CH
    elems = CH * D
    steps = elems // (UNROLL * _LANES)
    mesh = plsc.VectorSubcoreMesh(core_axis_name="c", subcore_axis_name="s")

    @functools.partial(
        pl.kernel,
        out_type=jax.ShapeDtypeStruct((B * S * D,), jnp.float32),
        mesh=mesh,
        scratch_types=[
            pltpu.VMEM((elems,), jnp.float32),
            pltpu.VMEM((elems,), jnp.float32),
        ],
    )
    def sc_add(x_hbm, pos_hbm, out_hbm, posb, xbuf):
        wid = lax.axis_index("s") * NC + lax.axis_index("c")
        sbase = wid * rows_per_w

        for c in range(n_chunks):
            prow = sbase + c * CH
            pltpu.sync_copy(pos_hbm.at[pl.ds(prow * D, elems)], posb)
            for b in range(B):
                xoff = (b * S + prow) * D
                pltpu.sync_copy(x_hbm.at[pl.ds(xoff, elems)], xbuf)

                def addstep(j, carry):
                    base = j * (UNROLL * _LANES)
                    for u in range(UNROLL):
                        o = base + u * _LANES
                        plsc.addupdate(xbuf.at[pl.ds(o, _LANES)],
                                       posb[pl.ds(o, _LANES)])
                    return carry

                lax.fori_loop(0, steps, addstep, 0)
                pltpu.sync_copy(xbuf, out_hbm.at[pl.ds(xoff, elems)])

    return sc_add


def kernel(x, pos_table):
    B, S, D = x.shape
    info = plsc.get_sparse_core_info()
    NC, NS = info.num_cores, info.num_subcores
    out = _make_sc_add(B, S, D, NC, NS, CH=32, UNROLL=16)(
        x.reshape(-1), pos_table[:S].reshape(-1))
    return out.reshape(B, S, D)
